# baseline (speedup 1.0000x reference)
import functools

import jax
import jax.numpy as jnp
from jax.experimental import pallas as pl
from jax.experimental.pallas import tpu as pltpu

N = 50000
E = 800000
H = 64
G = 32
OUT = 6
EPS = 1e-5

BN = 2000
BE = 8000


def _enc_body(x_ref, w_ref, b_ref, o_ref):
    o_ref[...] = (
        jnp.dot(x_ref[...], w_ref[...], preferred_element_type=jnp.float32)
        + b_ref[...]
    )


def _encode(x, enc_W, enc_b):
    return pl.pallas_call(
        _enc_body,
        grid=(N // BN,),
        in_specs=[
            pl.BlockSpec((BN, x.shape[1]), lambda i: (i, 0)),
            pl.BlockSpec((x.shape[1], H), lambda i: (0, 0)),
            pl.BlockSpec((1, H), lambda i: (0, 0)),
        ],
        out_specs=pl.BlockSpec((BN, H), lambda i: (i, 0)),
        out_shape=jax.ShapeDtypeStruct((N, H), jnp.float32),
    )(x, enc_W, enc_b.reshape(1, H))


def _pre_body(h_ref, w_ref, b_ref, o_ref):
    o_ref[...] = (
        jnp.dot(h_ref[...], w_ref[...], preferred_element_type=jnp.float32)
        + b_ref[...]
    )


def _pre(h, W1ab, b1ab):
    return pl.pallas_call(
        _pre_body,
        grid=(N // BN,),
        in_specs=[
            pl.BlockSpec((BN, H), lambda i: (i, 0)),
            pl.BlockSpec((H, 2 * H), lambda i: (0, 0)),
            pl.BlockSpec((1, 2 * H), lambda i: (0, 0)),
        ],
        out_specs=pl.BlockSpec((BN, 2 * H), lambda i: (i, 0)),
        out_shape=jax.ShapeDtypeStruct((N, 2 * H), jnp.float32),
    )(h, W1ab, b1ab)


def _edge_body(xa_ref, xb_ref, ea_ref, w_ref, o_ref):
    t = xa_ref[...] + xb_ref[...] + ea_ref[...] * w_ref[...]
    o_ref[...] = t * jax.nn.sigmoid(t)


def _edge_silu(xa, xb, ea, w_ea):
    return pl.pallas_call(
        _edge_body,
        grid=(E // BE,),
        in_specs=[
            pl.BlockSpec((BE, H), lambda i: (i, 0)),
            pl.BlockSpec((BE, H), lambda i: (i, 0)),
            pl.BlockSpec((BE, 1), lambda i: (i, 0)),
            pl.BlockSpec((1, H), lambda i: (0, 0)),
        ],
        out_specs=pl.BlockSpec((BE, H), lambda i: (i, 0)),
        out_shape=jax.ShapeDtypeStruct((E, H), jnp.float32),
    )(xa, xb, ea, w_ea.reshape(1, H))


def _post_body(h_ref, s_ref, deg_ref, w2_ref, b2_ref, g_ref, b_ref, o_ref):
    t = (
        h_ref[...]
        + jnp.dot(s_ref[...], w2_ref[...], preferred_element_type=jnp.float32)
        + deg_ref[...] * b2_ref[...]
    )
    mu = jnp.mean(t, axis=1, keepdims=True)
    var = jnp.mean((t - mu) ** 2, axis=1, keepdims=True)
    o_ref[...] = (t - mu) * jax.lax.rsqrt(var + EPS) * g_ref[...] + b_ref[...]


def _post(h, S, deg, W2, b2, ln_g, ln_b):
    return pl.pallas_call(
        _post_body,
        grid=(N // BN,),
        in_specs=[
            pl.BlockSpec((BN, H), lambda i: (i, 0)),
            pl.BlockSpec((BN, H), lambda i: (i, 0)),
            pl.BlockSpec((BN, 1), lambda i: (i, 0)),
            pl.BlockSpec((H, H), lambda i: (0, 0)),
            pl.BlockSpec((1, H), lambda i: (0, 0)),
            pl.BlockSpec((1, H), lambda i: (0, 0)),
            pl.BlockSpec((1, H), lambda i: (0, 0)),
        ],
        out_specs=pl.BlockSpec((BN, H), lambda i: (i, 0)),
        out_shape=jax.ShapeDtypeStruct((N, H), jnp.float32),
    )(h, S, deg, W2, b2.reshape(1, H), ln_g.reshape(1, H), ln_b.reshape(1, H))


def _pool_body(h_ref, bat_ref, h1w_ref, h1b_ref, h2w_ref, h2b_ref, o_ref,
               acc_ref, cnt_ref):
    i = pl.program_id(0)
    nb = pl.num_programs(0)

    @pl.when(i == 0)
    def _init():
        acc_ref[...] = jnp.zeros_like(acc_ref)
        cnt_ref[...] = jnp.zeros_like(cnt_ref)

    gid = jax.lax.broadcasted_iota(jnp.int32, (BN, G), 1)
    oh = (bat_ref[...] == gid).astype(jnp.float32)
    acc_ref[...] += jax.lax.dot_general(
        oh, h_ref[...], (((0,), (0,)), ((), ())),
        preferred_element_type=jnp.float32)
    cnt_ref[...] += jax.lax.dot_general(
        oh, jnp.ones((BN, 8), jnp.float32), (((0,), (0,)), ((), ())),
        preferred_element_type=jnp.float32)

    @pl.when(i == nb - 1)
    def _final():
        cnt = cnt_ref[...][:, 0:1]
        pooled = acc_ref[...] / jnp.maximum(cnt, 1.0)
        z = (
            jnp.dot(pooled, h1w_ref[...], preferred_element_type=jnp.float32)
            + h1b_ref[...]
        )
        z = z * jax.nn.sigmoid(z)
        o_ref[...] = (
            jnp.dot(z, h2w_ref[...], preferred_element_type=jnp.float32)
            + h2b_ref[...]
        )


def _pool_head(h, batch, head1_W, head1_b, head2_W, head2_b):
    return pl.pallas_call(
        _pool_body,
        grid=(N // BN,),
        in_specs=[
            pl.BlockSpec((BN, H), lambda i: (i, 0)),
            pl.BlockSpec((BN, 1), lambda i: (i, 0)),
            pl.BlockSpec((H, H), lambda i: (0, 0)),
            pl.BlockSpec((1, H), lambda i: (0, 0)),
            pl.BlockSpec((H, OUT), lambda i: (0, 0)),
            pl.BlockSpec((1, OUT), lambda i: (0, 0)),
        ],
        out_specs=pl.BlockSpec((G, OUT), lambda i: (0, 0)),
        out_shape=jax.ShapeDtypeStruct((G, OUT), jnp.float32),
        scratch_shapes=[
            pltpu.VMEM((G, H), jnp.float32),
            pltpu.VMEM((G, 8), jnp.float32),
        ],
    )(h, batch.reshape(N, 1), head1_W, head1_b.reshape(1, H),
      head2_W, head2_b.reshape(1, OUT))


def kernel(x, edge_index, edge_attr, batch, enc_W, enc_b, mlp1_W, mlp1_b,
           mlp2_W, mlp2_b, ln_g, ln_b, head1_W, head1_b, head2_W, head2_b):
    src = edge_index[0]
    dst = edge_index[1]
    L = mlp1_W.shape[0]

    h = _encode(x, enc_W, enc_b)
    deg = jax.ops.segment_sum(
        jnp.ones((E,), jnp.float32), dst, num_segments=N
    ).reshape(N, 1)

    for l in range(L):
        W1 = mlp1_W[l]
        W1ab = jnp.concatenate([W1[:H], W1[H:2 * H]], axis=1)
        b1ab = jnp.concatenate(
            [mlp1_b[l], jnp.zeros((H,), jnp.float32)]).reshape(1, 2 * H)
        w_ea = W1[2 * H]

        AB = _pre(h, W1ab, b1ab)
        xa = jnp.take(AB[:, :H], dst, axis=0)
        xb = jnp.take(AB[:, H:], src, axis=0)
        s = _edge_silu(xa, xb, edge_attr, w_ea)
        S = jax.ops.segment_sum(s, dst, num_segments=N)
        h = _post(h, S, deg, mlp2_W[l], mlp2_b[l], ln_g[l], ln_b[l])

    return _pool_head(h, batch, head1_W, head1_b, head2_W, head2_b)


# speedup vs baseline: 3.6696x; 3.6696x over previous
import functools

import jax
import jax.numpy as jnp
from jax.experimental import pallas as pl
from jax.experimental.pallas import tpu as pltpu
from jax.experimental.pallas import tpu_sc as plsc

N = 50000
E = 800000
H = 64
G = 32
OUT = 6
EPS = 1e-5

BN = 2000

HQ = 16
KE = 400
EPT = E // 16
NCH = EPT // KE

_SC_MESH = plsc.VectorSubcoreMesh(core_axis_name="c", subcore_axis_name="s")


def _enc_body(x_ref, w_ref, b_ref, o_ref):
    o_ref[...] = (
        jnp.dot(x_ref[...], w_ref[...], preferred_element_type=jnp.float32)
        + b_ref[...]
    )


def _encode(x, enc_W, enc_b):
    return pl.pallas_call(
        _enc_body,
        grid=(N // BN,),
        in_specs=[
            pl.BlockSpec((BN, x.shape[1]), lambda i: (i, 0)),
            pl.BlockSpec((x.shape[1], H), lambda i: (0, 0)),
            pl.BlockSpec((1, H), lambda i: (0, 0)),
        ],
        out_specs=pl.BlockSpec((BN, H), lambda i: (i, 0)),
        out_shape=jax.ShapeDtypeStruct((N, H), jnp.float32),
    )(x, enc_W, enc_b.reshape(1, H))


def _pre_body(h_ref, w_ref, b_ref, oa_ref, ob_ref):
    ab = (
        jnp.dot(h_ref[...], w_ref[...], preferred_element_type=jnp.float32)
        + b_ref[...]
    )
    for q in range(4):
        oa_ref[q] = ab[:, HQ * q:HQ * (q + 1)]
        ob_ref[q] = ab[:, H + HQ * q:H + HQ * (q + 1)]


def _pre(h, W1ab, b1ab):
    return pl.pallas_call(
        _pre_body,
        grid=(N // BN,),
        in_specs=[
            pl.BlockSpec((BN, H), lambda i: (i, 0)),
            pl.BlockSpec((H, 2 * H), lambda i: (0, 0)),
            pl.BlockSpec((1, 2 * H), lambda i: (0, 0)),
        ],
        out_specs=[
            pl.BlockSpec((4, BN, HQ), lambda i: (0, i, 0)),
            pl.BlockSpec((4, BN, HQ), lambda i: (0, i, 0)),
        ],
        out_shape=[
            jax.ShapeDtypeStruct((4, N, HQ), jnp.float32),
            jax.ShapeDtypeStruct((4, N, HQ), jnp.float32),
        ],
    )(h, W1ab, b1ab)


def _post_body(h_ref, s0_ref, s1_ref, s2_ref, s3_ref, deg_ref, w2_ref,
               b2_ref, g_ref, b_ref, o_ref):
    w2 = w2_ref[...]
    t = h_ref[...] + deg_ref[...] * b2_ref[...]
    for q, s_ref in enumerate((s0_ref, s1_ref, s2_ref, s3_ref)):
        t += jnp.dot(s_ref[...], w2[HQ * q:HQ * (q + 1), :],
                     preferred_element_type=jnp.float32)
    mu = jnp.mean(t, axis=1, keepdims=True)
    var = jnp.mean((t - mu) ** 2, axis=1, keepdims=True)
    o_ref[...] = (t - mu) * jax.lax.rsqrt(var + EPS) * g_ref[...] + b_ref[...]


def _post(h, Sflat, deg, W2, b2, ln_g, ln_b):
    nb = N // BN
    return pl.pallas_call(
        _post_body,
        grid=(nb,),
        in_specs=[
            pl.BlockSpec((BN, H), lambda i: (i, 0)),
            pl.BlockSpec((BN, HQ), lambda i: (i, 0)),
            pl.BlockSpec((BN, HQ), lambda i: (i + nb, 0)),
            pl.BlockSpec((BN, HQ), lambda i: (i + 2 * nb, 0)),
            pl.BlockSpec((BN, HQ), lambda i: (i + 3 * nb, 0)),
            pl.BlockSpec((BN, 1), lambda i: (i, 0)),
            pl.BlockSpec((H, H), lambda i: (0, 0)),
            pl.BlockSpec((1, H), lambda i: (0, 0)),
            pl.BlockSpec((1, H), lambda i: (0, 0)),
            pl.BlockSpec((1, H), lambda i: (0, 0)),
        ],
        out_specs=pl.BlockSpec((BN, H), lambda i: (i, 0)),
        out_shape=jax.ShapeDtypeStruct((N, H), jnp.float32),
    )(h, Sflat, Sflat, Sflat, Sflat, deg, W2, b2.reshape(1, H),
      ln_g.reshape(1, H), ln_b.reshape(1, H))


def _pool_body(h_ref, bat_ref, h1w_ref, h1b_ref, h2w_ref, h2b_ref, o_ref,
               acc_ref, cnt_ref):
    i = pl.program_id(0)
    nb = pl.num_programs(0)

    @pl.when(i == 0)
    def _init():
        acc_ref[...] = jnp.zeros_like(acc_ref)
        cnt_ref[...] = jnp.zeros_like(cnt_ref)

    gid = jax.lax.broadcasted_iota(jnp.int32, (BN, G), 1)
    oh = (bat_ref[...] == gid).astype(jnp.float32)
    acc_ref[...] += jax.lax.dot_general(
        oh, h_ref[...], (((0,), (0,)), ((), ())),
        preferred_element_type=jnp.float32)
    cnt_ref[...] += jax.lax.dot_general(
        oh, jnp.ones((BN, 8), jnp.float32), (((0,), (0,)), ((), ())),
        preferred_element_type=jnp.float32)

    @pl.when(i == nb - 1)
    def _final():
        cnt = cnt_ref[...][:, 0:1]
        pooled = acc_ref[...] / jnp.maximum(cnt, 1.0)
        z = (
            jnp.dot(pooled, h1w_ref[...], preferred_element_type=jnp.float32)
            + h1b_ref[...]
        )
        z = z * jax.nn.sigmoid(z)
        o_ref[...] = (
            jnp.dot(z, h2w_ref[...], preferred_element_type=jnp.float32)
            + h2b_ref[...]
        )


def _pool_head(h, batch, head1_W, head1_b, head2_W, head2_b):
    return pl.pallas_call(
        _pool_body,
        grid=(N // BN,),
        in_specs=[
            pl.BlockSpec((BN, H), lambda i: (i, 0)),
            pl.BlockSpec((BN, 1), lambda i: (i, 0)),
            pl.BlockSpec((H, H), lambda i: (0, 0)),
            pl.BlockSpec((1, H), lambda i: (0, 0)),
            pl.BlockSpec((H, OUT), lambda i: (0, 0)),
            pl.BlockSpec((1, OUT), lambda i: (0, 0)),
        ],
        out_specs=pl.BlockSpec((G, OUT), lambda i: (0, 0)),
        out_shape=jax.ShapeDtypeStruct((G, OUT), jnp.float32),
        scratch_shapes=[
            pltpu.VMEM((G, H), jnp.float32),
            pltpu.VMEM((G, 8), jnp.float32),
        ],
    )(h, batch.reshape(N, 1), head1_W, head1_b.reshape(1, H),
      head2_W, head2_b.reshape(1, OUT))


@functools.partial(
    pl.kernel,
    mesh=_SC_MESH,
    compiler_params=pltpu.CompilerParams(use_tc_tiling_on_sc=False),
    out_type=jax.ShapeDtypeStruct((4 * N, HQ), jnp.float32),
    scratch_types=[
        pltpu.VMEM((KE,), jnp.int32),
        pltpu.VMEM((KE,), jnp.int32),
        pltpu.VMEM((KE,), jnp.int32),
        pltpu.VMEM((KE,), jnp.float32),
        pltpu.VMEM((KE, HQ), jnp.float32),
        pltpu.VMEM((KE, HQ), jnp.float32),
        pltpu.VMEM((KE, HQ), jnp.float32),
        pltpu.VMEM((4, HQ), jnp.float32),
        pltpu.VMEM_SHARED((N, HQ), jnp.float32),
        pltpu.SemaphoreType.DMA,
        pltpu.SemaphoreType.DMA,
    ],
)
def _sc_edge(A_hbm, B_hbm, dst_hbm, src_hbm, ea_hbm, w_hbm, out_hbm,
             dstv, srcv, idxA, eav, gA, gB, sbuf, wv, acc, sem1, sem2):
    c = jax.lax.axis_index("c")
    s = jax.lax.axis_index("s")
    pltpu.sync_copy(w_hbm, wv)

    zero = jnp.zeros((16,), jnp.float32)
    npz = N // KE

    for p in range(2):
        q = 2 * c + p
        qN = q * N
        w0 = wv[q, 0:16]

        def zrow(r, carry):
            sbuf[r, 0:16] = zero
            return carry
        jax.lax.fori_loop(0, KE, zrow, 0)

        def zacc(k, carry):
            pz = s + 16 * k

            @pl.when(pz < npz)
            def _():
                pltpu.sync_copy(sbuf, acc.at[pl.ds(pz * KE, KE)])
            return carry
        jax.lax.fori_loop(0, (npz + 15) // 16, zacc, 0)

        plsc.subcore_barrier()

        def chunk(ch, carry):
            base = s * EPT + ch * KE
            pltpu.sync_copy(dst_hbm.at[pl.ds(base, KE)], dstv)
            pltpu.sync_copy(src_hbm.at[pl.ds(base, KE)], srcv)
            pltpu.sync_copy(ea_hbm.at[pl.ds(base, KE)], eav)

            def offs(r, cr):
                sl = pl.ds(r * 16, 16)
                idxA[sl] = dstv[sl] + qN
                srcv[sl] = srcv[sl] + qN
                return cr
            jax.lax.fori_loop(0, KE // 16, offs, 0)

            cpA = pltpu.async_copy(A_hbm.at[idxA], gA, sem1)
            cpB = pltpu.async_copy(B_hbm.at[srcv], gB, sem2)
            cpA.wait()
            cpB.wait()

            def edge16(g, cr):
                e0 = g * 16
                ev = eav[pl.ds(e0, 16)]
                for j in range(16):
                    e = e0 + j
                    t = gA[e, 0:16] + gB[e, 0:16] + ev[j] * w0
                    sbuf[e, 0:16] = t / (1.0 + jnp.exp(-t))
                return cr
            jax.lax.fori_loop(0, KE // 16, edge16, 0)

            pltpu.sync_copy(sbuf, acc.at[dstv], add=True)
            return carry
        jax.lax.fori_loop(0, NCH, chunk, 0)

        plsc.subcore_barrier()

        def wout(k, carry):
            pz = s + 16 * k

            @pl.when(pz < npz)
            def _():
                pltpu.sync_copy(acc.at[pl.ds(pz * KE, KE)],
                                out_hbm.at[pl.ds(qN + pz * KE, KE)])
            return carry
        jax.lax.fori_loop(0, (npz + 15) // 16, wout, 0)

        plsc.subcore_barrier()


def kernel(x, edge_index, edge_attr, batch, enc_W, enc_b, mlp1_W, mlp1_b,
           mlp2_W, mlp2_b, ln_g, ln_b, head1_W, head1_b, head2_W, head2_b):
    src = edge_index[0]
    dst = edge_index[1]
    ea = edge_attr.reshape(E)
    L = mlp1_W.shape[0]

    h = _encode(x, enc_W, enc_b)
    deg = jax.ops.segment_sum(
        jnp.ones((E,), jnp.float32), dst, num_segments=N
    ).reshape(N, 1)

    for l in range(L):
        W1 = mlp1_W[l]
        W1ab = jnp.concatenate([W1[:H], W1[H:2 * H]], axis=1)
        b1ab = jnp.concatenate(
            [mlp1_b[l], jnp.zeros((H,), jnp.float32)]).reshape(1, 2 * H)
        w_st = W1[2 * H].reshape(4, HQ)

        Ast, Bst = _pre(h, W1ab, b1ab)
        Sflat = _sc_edge(Ast.reshape(4 * N, HQ), Bst.reshape(4 * N, HQ),
                         dst, src, ea, w_st)
        h = _post(h, Sflat, deg, mlp2_W[l], mlp2_b[l], ln_g[l], ln_b[l])

    return _pool_head(h, batch, head1_W, head1_b, head2_W, head2_b)


# speedup vs baseline: 5.8738x; 1.6007x over previous
import functools

import jax
import jax.numpy as jnp
from jax.experimental import pallas as pl
from jax.experimental.pallas import tpu as pltpu
from jax.experimental.pallas import tpu_sc as plsc

N = 50000
E = 800000
H = 64
G = 32
OUT = 6
EPS = 1e-5

BN = 2000

HQ = 16
KE = 400
EPT = E // 16
NCH = EPT // KE

_SC_MESH = plsc.VectorSubcoreMesh(core_axis_name="c", subcore_axis_name="s")


def _enc_body(x_ref, w_ref, b_ref, o_ref):
    o_ref[...] = (
        jnp.dot(x_ref[...], w_ref[...], preferred_element_type=jnp.float32)
        + b_ref[...]
    )


def _encode(x, enc_W, enc_b):
    return pl.pallas_call(
        _enc_body,
        grid=(N // BN,),
        in_specs=[
            pl.BlockSpec((BN, x.shape[1]), lambda i: (i, 0)),
            pl.BlockSpec((x.shape[1], H), lambda i: (0, 0)),
            pl.BlockSpec((1, H), lambda i: (0, 0)),
        ],
        out_specs=pl.BlockSpec((BN, H), lambda i: (i, 0)),
        out_shape=jax.ShapeDtypeStruct((N, H), jnp.float32),
    )(x, enc_W, enc_b.reshape(1, H))


def _pre_body(h_ref, w_ref, b_ref, oa_ref, ob_ref):
    ab = (
        jnp.dot(h_ref[...], w_ref[...], preferred_element_type=jnp.float32)
        + b_ref[...]
    )
    for q in range(4):
        oa_ref[q] = ab[:, HQ * q:HQ * (q + 1)]
        ob_ref[q] = ab[:, H + HQ * q:H + HQ * (q + 1)]


def _pre(h, W1ab, b1ab):
    return pl.pallas_call(
        _pre_body,
        grid=(N // BN,),
        in_specs=[
            pl.BlockSpec((BN, H), lambda i: (i, 0)),
            pl.BlockSpec((H, 2 * H), lambda i: (0, 0)),
            pl.BlockSpec((1, 2 * H), lambda i: (0, 0)),
        ],
        out_specs=[
            pl.BlockSpec((4, BN, HQ), lambda i: (0, i, 0)),
            pl.BlockSpec((4, BN, HQ), lambda i: (0, i, 0)),
        ],
        out_shape=[
            jax.ShapeDtypeStruct((4, N, HQ), jnp.float32),
            jax.ShapeDtypeStruct((4, N, HQ), jnp.float32),
        ],
    )(h, W1ab, b1ab)


def _post_body(h_ref, s0_ref, s1_ref, s2_ref, s3_ref, deg_ref, w2_ref,
               b2_ref, g_ref, b_ref, o_ref):
    w2 = w2_ref[...]
    t = h_ref[...] + deg_ref[...] * b2_ref[...]
    for q, s_ref in enumerate((s0_ref, s1_ref, s2_ref, s3_ref)):
        t += jnp.dot(s_ref[...], w2[HQ * q:HQ * (q + 1), :],
                     preferred_element_type=jnp.float32)
    mu = jnp.mean(t, axis=1, keepdims=True)
    var = jnp.mean((t - mu) ** 2, axis=1, keepdims=True)
    o_ref[...] = (t - mu) * jax.lax.rsqrt(var + EPS) * g_ref[...] + b_ref[...]


def _post(h, Sflat, deg, W2, b2, ln_g, ln_b):
    nb = N // BN
    return pl.pallas_call(
        _post_body,
        grid=(nb,),
        in_specs=[
            pl.BlockSpec((BN, H), lambda i: (i, 0)),
            pl.BlockSpec((BN, HQ), lambda i: (i, 0)),
            pl.BlockSpec((BN, HQ), lambda i: (i + nb, 0)),
            pl.BlockSpec((BN, HQ), lambda i: (i + 2 * nb, 0)),
            pl.BlockSpec((BN, HQ), lambda i: (i + 3 * nb, 0)),
            pl.BlockSpec((BN, 1), lambda i: (i, 0)),
            pl.BlockSpec((H, H), lambda i: (0, 0)),
            pl.BlockSpec((1, H), lambda i: (0, 0)),
            pl.BlockSpec((1, H), lambda i: (0, 0)),
            pl.BlockSpec((1, H), lambda i: (0, 0)),
        ],
        out_specs=pl.BlockSpec((BN, H), lambda i: (i, 0)),
        out_shape=jax.ShapeDtypeStruct((N, H), jnp.float32),
    )(h, Sflat, Sflat, Sflat, Sflat, deg, W2, b2.reshape(1, H),
      ln_g.reshape(1, H), ln_b.reshape(1, H))


def _pool_body(h_ref, bat_ref, h1w_ref, h1b_ref, h2w_ref, h2b_ref, o_ref,
               acc_ref, cnt_ref):
    i = pl.program_id(0)
    nb = pl.num_programs(0)

    @pl.when(i == 0)
    def _init():
        acc_ref[...] = jnp.zeros_like(acc_ref)
        cnt_ref[...] = jnp.zeros_like(cnt_ref)

    gid = jax.lax.broadcasted_iota(jnp.int32, (BN, G), 1)
    oh = (bat_ref[...] == gid).astype(jnp.float32)
    acc_ref[...] += jax.lax.dot_general(
        oh, h_ref[...], (((0,), (0,)), ((), ())),
        preferred_element_type=jnp.float32)
    cnt_ref[...] += jax.lax.dot_general(
        oh, jnp.ones((BN, 8), jnp.float32), (((0,), (0,)), ((), ())),
        preferred_element_type=jnp.float32)

    @pl.when(i == nb - 1)
    def _final():
        cnt = cnt_ref[...][:, 0:1]
        pooled = acc_ref[...] / jnp.maximum(cnt, 1.0)
        z = (
            jnp.dot(pooled, h1w_ref[...], preferred_element_type=jnp.float32)
            + h1b_ref[...]
        )
        z = z * jax.nn.sigmoid(z)
        o_ref[...] = (
            jnp.dot(z, h2w_ref[...], preferred_element_type=jnp.float32)
            + h2b_ref[...]
        )


def _pool_head(h, batch, head1_W, head1_b, head2_W, head2_b):
    return pl.pallas_call(
        _pool_body,
        grid=(N // BN,),
        in_specs=[
            pl.BlockSpec((BN, H), lambda i: (i, 0)),
            pl.BlockSpec((BN, 1), lambda i: (i, 0)),
            pl.BlockSpec((H, H), lambda i: (0, 0)),
            pl.BlockSpec((1, H), lambda i: (0, 0)),
            pl.BlockSpec((H, OUT), lambda i: (0, 0)),
            pl.BlockSpec((1, OUT), lambda i: (0, 0)),
        ],
        out_specs=pl.BlockSpec((G, OUT), lambda i: (0, 0)),
        out_shape=jax.ShapeDtypeStruct((G, OUT), jnp.float32),
        scratch_shapes=[
            pltpu.VMEM((G, H), jnp.float32),
            pltpu.VMEM((G, 8), jnp.float32),
        ],
    )(h, batch.reshape(N, 1), head1_W, head1_b.reshape(1, H),
      head2_W, head2_b.reshape(1, OUT))


@functools.partial(
    pl.kernel,
    mesh=_SC_MESH,
    compiler_params=pltpu.CompilerParams(use_tc_tiling_on_sc=False),
    out_type=jax.ShapeDtypeStruct((4 * N, HQ), jnp.float32),
    scratch_types=[
        pltpu.VMEM((2, KE), jnp.int32),
        pltpu.VMEM((2, KE), jnp.int32),
        pltpu.VMEM((2, KE), jnp.int32),
        pltpu.VMEM((2, KE), jnp.float32),
        pltpu.VMEM((2, KE, HQ), jnp.float32),
        pltpu.VMEM((2, KE, HQ), jnp.float32),
        pltpu.VMEM((KE, HQ), jnp.float32),
        pltpu.VMEM((4, HQ), jnp.float32),
        pltpu.VMEM_SHARED((N, HQ), jnp.float32),
        pltpu.SemaphoreType.DMA,
        pltpu.SemaphoreType.DMA,
        pltpu.SemaphoreType.DMA,
        pltpu.SemaphoreType.DMA,
        pltpu.SemaphoreType.DMA,
        pltpu.SemaphoreType.DMA,
        pltpu.SemaphoreType.DMA,
        pltpu.SemaphoreType.DMA,
    ],
)
def _sc_edge(A_hbm, B_hbm, dstq_hbm, srcq_hbm, dst_hbm, ea_hbm, w_hbm,
             out_hbm, iqa, iqb, idv, iea, gA, gB, sbuf, wv, acc,
             semi0, semi1, semE0, semE1, semA0, semA1, semB0, semB1):
    c = jax.lax.axis_index("c")
    s = jax.lax.axis_index("s")
    pltpu.sync_copy(w_hbm, wv)

    semi = (semi0, semi1)
    semE = (semE0, semE1)
    semA = (semA0, semA1)
    semB = (semB0, semB1)

    zero = jnp.zeros((16,), jnp.float32)
    npz = N // KE

    for p in range(2):
        q = 2 * c + p
        qE = q * E
        w0 = wv[q, 0:16]

        def zrow(r, carry):
            sbuf[r, 0:16] = zero
            return carry
        jax.lax.fori_loop(0, KE, zrow, 0)

        def zacc(k, carry):
            pz = s + 16 * k

            @pl.when(pz < npz)
            def _():
                pltpu.sync_copy(sbuf, acc.at[pl.ds(pz * KE, KE)])
            return carry
        jax.lax.fori_loop(0, (npz + 15) // 16, zacc, 0)

        plsc.subcore_barrier()

        def issue_ab(ch, sl):
            base = s * EPT + ch * KE
            pltpu.async_copy(dstq_hbm.at[pl.ds(qE + base, KE)], iqa.at[sl],
                             semi[sl])
            pltpu.async_copy(srcq_hbm.at[pl.ds(qE + base, KE)], iqb.at[sl],
                             semi[sl])

        def wait_ab(sl):
            pltpu.make_async_copy(dstq_hbm.at[pl.ds(0, KE)], iqa.at[sl],
                                  semi[sl]).wait()
            pltpu.make_async_copy(srcq_hbm.at[pl.ds(0, KE)], iqb.at[sl],
                                  semi[sl]).wait()

        def issue_de(ch, sl):
            base = s * EPT + ch * KE
            pltpu.async_copy(dst_hbm.at[pl.ds(base, KE)], idv.at[sl],
                             semE[sl])
            pltpu.async_copy(ea_hbm.at[pl.ds(base, KE)], iea.at[sl], semE[sl])

        def wait_de(sl):
            pltpu.make_async_copy(dst_hbm.at[pl.ds(0, KE)], idv.at[sl],
                                  semE[sl]).wait()
            pltpu.make_async_copy(ea_hbm.at[pl.ds(0, KE)], iea.at[sl],
                                  semE[sl]).wait()

        def issue_gather(sl):
            pltpu.async_copy(A_hbm.at[iqa.at[sl]], gA.at[sl], semA[sl])
            pltpu.async_copy(B_hbm.at[iqb.at[sl]], gB.at[sl], semB[sl])

        def wait_gather(sl):
            pltpu.make_async_copy(A_hbm.at[iqa.at[sl]], gA.at[sl],
                                  semA[sl]).wait()
            pltpu.make_async_copy(B_hbm.at[iqb.at[sl]], gB.at[sl],
                                  semB[sl]).wait()

        def half_step(ch, sl):
            wait_gather(sl)

            @pl.when(ch + 2 < NCH)
            def _():
                issue_ab(ch + 2, sl)

            wait_de(sl)

            def edge16(g, cr):
                e0 = g * 16
                ev = iea[sl, pl.ds(e0, 16)]
                for j in range(16):
                    e = e0 + j
                    t = gA[sl, e, 0:16] + gB[sl, e, 0:16] + ev[j] * w0
                    sbuf[e, 0:16] = t / (1.0 + jnp.exp(-t))
                return cr
            jax.lax.fori_loop(0, KE // 16, edge16, 0)

            pltpu.sync_copy(sbuf, acc.at[idv.at[sl]], add=True)

            @pl.when(ch + 2 < NCH)
            def _():
                issue_de(ch + 2, sl)
                wait_ab(sl)
                issue_gather(sl)

        issue_ab(0, 0)
        issue_de(0, 0)
        issue_ab(1, 1)
        issue_de(1, 1)
        wait_ab(0)
        issue_gather(0)
        wait_ab(1)
        issue_gather(1)

        def pair(i, carry):
            half_step(2 * i, 0)
            half_step(2 * i + 1, 1)
            return carry
        jax.lax.fori_loop(0, NCH // 2, pair, 0)
        if NCH % 2:
            half_step(NCH - 1, 0)

        plsc.subcore_barrier()

        def wout(k, carry):
            pz = s + 16 * k

            @pl.when(pz < npz)
            def _():
                pltpu.sync_copy(acc.at[pl.ds(pz * KE, KE)],
                                out_hbm.at[pl.ds(q * N + pz * KE, KE)])
            return carry
        jax.lax.fori_loop(0, (npz + 15) // 16, wout, 0)

        plsc.subcore_barrier()


def kernel(x, edge_index, edge_attr, batch, enc_W, enc_b, mlp1_W, mlp1_b,
           mlp2_W, mlp2_b, ln_g, ln_b, head1_W, head1_b, head2_W, head2_b):
    src = edge_index[0]
    dst = edge_index[1]
    ea = edge_attr.reshape(E)
    L = mlp1_W.shape[0]

    h = _encode(x, enc_W, enc_b)
    deg = jax.ops.segment_sum(
        jnp.ones((E,), jnp.float32), dst, num_segments=N
    ).reshape(N, 1)
    qoff = (jnp.arange(4, dtype=jnp.int32) * N)[:, None]
    dstq = (dst[None, :] + qoff).reshape(4 * E)
    srcq = (src[None, :] + qoff).reshape(4 * E)

    for l in range(L):
        W1 = mlp1_W[l]
        W1ab = jnp.concatenate([W1[:H], W1[H:2 * H]], axis=1)
        b1ab = jnp.concatenate(
            [mlp1_b[l], jnp.zeros((H,), jnp.float32)]).reshape(1, 2 * H)
        w_st = W1[2 * H].reshape(4, HQ)

        Ast, Bst = _pre(h, W1ab, b1ab)
        Sflat = _sc_edge(Ast.reshape(4 * N, HQ), Bst.reshape(4 * N, HQ),
                         dstq, srcq, dst, ea, w_st)
        h = _post(h, Sflat, deg, mlp2_W[l], mlp2_b[l], ln_g[l], ln_b[l])

    return _pool_head(h, batch, head1_W, head1_b, head2_W, head2_b)


# speedup vs baseline: 8.1796x; 1.3925x over previous
import functools

import jax
import jax.numpy as jnp
from jax.experimental import pallas as pl
from jax.experimental.pallas import tpu as pltpu
from jax.experimental.pallas import tpu_sc as plsc

N = 50000
E = 800000
H = 64
G = 32
OUT = 6
EPS = 1e-5

BN = 2000

HQ = 16
KE = 400
EPT = E // 16
NCH = EPT // KE

_SC_MESH = plsc.VectorSubcoreMesh(core_axis_name="c", subcore_axis_name="s")


def _write_quarters(ab, oa_ref, ob_ref):
    for q in range(4):
        oa_ref[q] = ab[:, HQ * q:HQ * (q + 1)]
        ob_ref[q] = ab[:, H + HQ * q:H + HQ * (q + 1)]


def _enc_pre_body(x_ref, ew_ref, eb_ref, w_ref, b_ref, oh_ref, oa_ref,
                  ob_ref):
    hb = (
        jnp.dot(x_ref[...], ew_ref[...], preferred_element_type=jnp.float32)
        + eb_ref[...]
    )
    oh_ref[...] = hb
    ab = jnp.dot(hb, w_ref[...], preferred_element_type=jnp.float32) + b_ref[...]
    _write_quarters(ab, oa_ref, ob_ref)


def _enc_pre(x, enc_W, enc_b, W1ab, b1ab):
    f_in = x.shape[1]
    return pl.pallas_call(
        _enc_pre_body,
        grid=(N // BN,),
        in_specs=[
            pl.BlockSpec((BN, f_in), lambda i: (i, 0)),
            pl.BlockSpec((f_in, H), lambda i: (0, 0)),
            pl.BlockSpec((1, H), lambda i: (0, 0)),
            pl.BlockSpec((H, 2 * H), lambda i: (0, 0)),
            pl.BlockSpec((1, 2 * H), lambda i: (0, 0)),
        ],
        out_specs=[
            pl.BlockSpec((BN, H), lambda i: (i, 0)),
            pl.BlockSpec((4, BN, HQ), lambda i: (0, i, 0)),
            pl.BlockSpec((4, BN, HQ), lambda i: (0, i, 0)),
        ],
        out_shape=[
            jax.ShapeDtypeStruct((N, H), jnp.float32),
            jax.ShapeDtypeStruct((4, N, HQ), jnp.float32),
            jax.ShapeDtypeStruct((4, N, HQ), jnp.float32),
        ],
    )(x, enc_W, enc_b.reshape(1, H), W1ab, b1ab)


def _ln_update(h_ref, s_refs, deg_ref, w2_ref, b2_ref, g_ref, b_ref):
    w2 = w2_ref[...]
    t = h_ref[...] + deg_ref[...] * b2_ref[...]
    for q, s_ref in enumerate(s_refs):
        t += jnp.dot(s_ref[...], w2[HQ * q:HQ * (q + 1), :],
                     preferred_element_type=jnp.float32)
    mu = jnp.mean(t, axis=1, keepdims=True)
    var = jnp.mean((t - mu) ** 2, axis=1, keepdims=True)
    return (t - mu) * jax.lax.rsqrt(var + EPS) * g_ref[...] + b_ref[...]


def _post_pre_body(h_ref, s0_ref, s1_ref, s2_ref, s3_ref, deg_ref, w2_ref,
                   b2_ref, g_ref, b_ref, w1n_ref, b1n_ref, oh_ref, oa_ref,
                   ob_ref):
    hn = _ln_update(h_ref, (s0_ref, s1_ref, s2_ref, s3_ref), deg_ref, w2_ref,
                    b2_ref, g_ref, b_ref)
    oh_ref[...] = hn
    ab = (
        jnp.dot(hn, w1n_ref[...], preferred_element_type=jnp.float32)
        + b1n_ref[...]
    )
    _write_quarters(ab, oa_ref, ob_ref)


def _post_pre(h, Sflat, deg, W2, b2, ln_g, ln_b, W1n, b1n):
    nb = N // BN
    return pl.pallas_call(
        _post_pre_body,
        grid=(nb,),
        in_specs=[
            pl.BlockSpec((BN, H), lambda i: (i, 0)),
            pl.BlockSpec((BN, HQ), lambda i: (i, 0)),
            pl.BlockSpec((BN, HQ), lambda i: (i + nb, 0)),
            pl.BlockSpec((BN, HQ), lambda i: (i + 2 * nb, 0)),
            pl.BlockSpec((BN, HQ), lambda i: (i + 3 * nb, 0)),
            pl.BlockSpec((BN, 1), lambda i: (i, 0)),
            pl.BlockSpec((H, H), lambda i: (0, 0)),
            pl.BlockSpec((1, H), lambda i: (0, 0)),
            pl.BlockSpec((1, H), lambda i: (0, 0)),
            pl.BlockSpec((1, H), lambda i: (0, 0)),
            pl.BlockSpec((H, 2 * H), lambda i: (0, 0)),
            pl.BlockSpec((1, 2 * H), lambda i: (0, 0)),
        ],
        out_specs=[
            pl.BlockSpec((BN, H), lambda i: (i, 0)),
            pl.BlockSpec((4, BN, HQ), lambda i: (0, i, 0)),
            pl.BlockSpec((4, BN, HQ), lambda i: (0, i, 0)),
        ],
        out_shape=[
            jax.ShapeDtypeStruct((N, H), jnp.float32),
            jax.ShapeDtypeStruct((4, N, HQ), jnp.float32),
            jax.ShapeDtypeStruct((4, N, HQ), jnp.float32),
        ],
    )(h, Sflat, Sflat, Sflat, Sflat, deg, W2, b2.reshape(1, H),
      ln_g.reshape(1, H), ln_b.reshape(1, H), W1n, b1n)


def _post_pool_body(h_ref, s0_ref, s1_ref, s2_ref, s3_ref, deg_ref, w2_ref,
                    b2_ref, g_ref, b_ref, bat_ref, h1w_ref, h1b_ref, h2w_ref,
                    h2b_ref, o_ref, acc_ref, cnt_ref):
    i = pl.program_id(0)
    nb = pl.num_programs(0)

    hn = _ln_update(h_ref, (s0_ref, s1_ref, s2_ref, s3_ref), deg_ref, w2_ref,
                    b2_ref, g_ref, b_ref)

    @pl.when(i == 0)
    def _init():
        acc_ref[...] = jnp.zeros_like(acc_ref)
        cnt_ref[...] = jnp.zeros_like(cnt_ref)

    gid = jax.lax.broadcasted_iota(jnp.int32, (BN, G), 1)
    oh = (bat_ref[...] == gid).astype(jnp.float32)
    acc_ref[...] += jax.lax.dot_general(
        oh, hn, (((0,), (0,)), ((), ())), preferred_element_type=jnp.float32)
    cnt_ref[...] += jax.lax.dot_general(
        oh, jnp.ones((BN, 8), jnp.float32), (((0,), (0,)), ((), ())),
        preferred_element_type=jnp.float32)

    @pl.when(i == nb - 1)
    def _final():
        cnt = cnt_ref[...][:, 0:1]
        pooled = acc_ref[...] / jnp.maximum(cnt, 1.0)
        z = (
            jnp.dot(pooled, h1w_ref[...], preferred_element_type=jnp.float32)
            + h1b_ref[...]
        )
        z = z * jax.nn.sigmoid(z)
        o_ref[...] = (
            jnp.dot(z, h2w_ref[...], preferred_element_type=jnp.float32)
            + h2b_ref[...]
        )


def _post_pool(h, Sflat, deg, W2, b2, ln_g, ln_b, batch, head1_W, head1_b,
               head2_W, head2_b):
    nb = N // BN
    return pl.pallas_call(
        _post_pool_body,
        grid=(nb,),
        in_specs=[
            pl.BlockSpec((BN, H), lambda i: (i, 0)),
            pl.BlockSpec((BN, HQ), lambda i: (i, 0)),
            pl.BlockSpec((BN, HQ), lambda i: (i + nb, 0)),
            pl.BlockSpec((BN, HQ), lambda i: (i + 2 * nb, 0)),
            pl.BlockSpec((BN, HQ), lambda i: (i + 3 * nb, 0)),
            pl.BlockSpec((BN, 1), lambda i: (i, 0)),
            pl.BlockSpec((H, H), lambda i: (0, 0)),
            pl.BlockSpec((1, H), lambda i: (0, 0)),
            pl.BlockSpec((1, H), lambda i: (0, 0)),
            pl.BlockSpec((1, H), lambda i: (0, 0)),
            pl.BlockSpec((BN, 1), lambda i: (i, 0)),
            pl.BlockSpec((H, H), lambda i: (0, 0)),
            pl.BlockSpec((1, H), lambda i: (0, 0)),
            pl.BlockSpec((H, OUT), lambda i: (0, 0)),
            pl.BlockSpec((1, OUT), lambda i: (0, 0)),
        ],
        out_specs=pl.BlockSpec((G, OUT), lambda i: (0, 0)),
        out_shape=jax.ShapeDtypeStruct((G, OUT), jnp.float32),
        scratch_shapes=[
            pltpu.VMEM((G, H), jnp.float32),
            pltpu.VMEM((G, 8), jnp.float32),
        ],
    )(h, Sflat, Sflat, Sflat, Sflat, deg, W2, b2.reshape(1, H),
      ln_g.reshape(1, H), ln_b.reshape(1, H), batch.reshape(N, 1),
      head1_W, head1_b.reshape(1, H), head2_W, head2_b.reshape(1, OUT))


def _make_sc_edge(with_deg):
    out_type = [jax.ShapeDtypeStruct((4 * N, HQ), jnp.float32)]
    scratch = [
        pltpu.VMEM((2, KE), jnp.int32),
        pltpu.VMEM((2, KE), jnp.int32),
        pltpu.VMEM((2, KE), jnp.int32),
        pltpu.VMEM((2, KE), jnp.float32),
        pltpu.VMEM((2, KE, HQ), jnp.float32),
        pltpu.VMEM((2, KE, HQ), jnp.float32),
        pltpu.VMEM((KE, HQ), jnp.float32),
        pltpu.VMEM((4, HQ), jnp.float32),
        pltpu.VMEM_SHARED((N, HQ), jnp.float32),
    ]
    if with_deg:
        out_type.append(jax.ShapeDtypeStruct((N,), jnp.float32))
        scratch += [
            pltpu.VMEM((KE,), jnp.float32),
            pltpu.VMEM((KE,), jnp.float32),
            pltpu.VMEM_SHARED((N,), jnp.float32),
        ]
    scratch += [pltpu.SemaphoreType.DMA] * 8

    def body(A_hbm, B_hbm, dstq_hbm, srcq_hbm, dst_hbm, ea_hbm, w_hbm, *rest):
        if with_deg:
            (out_hbm, dout_hbm, iqa, iqb, idv, iea, gA, gB, sbuf, wv, acc,
             ones, zb1, dacc, semi0, semi1, semE0, semE1, semA0, semA1,
             semB0, semB1) = rest
        else:
            (out_hbm, iqa, iqb, idv, iea, gA, gB, sbuf, wv, acc,
             semi0, semi1, semE0, semE1, semA0, semA1, semB0, semB1) = rest
        c = jax.lax.axis_index("c")
        s = jax.lax.axis_index("s")
        pltpu.sync_copy(w_hbm, wv)

        semi = (semi0, semi1)
        semE = (semE0, semE1)
        semA = (semA0, semA1)
        semB = (semB0, semB1)

        zero = jnp.zeros((16,), jnp.float32)
        one = jnp.ones((16,), jnp.float32)
        npz = N // KE

        if with_deg:
            def fill1(r, carry):
                sl16 = pl.ds(r * 16, 16)
                ones[sl16] = one
                zb1[sl16] = zero
                return carry
            jax.lax.fori_loop(0, KE // 16, fill1, 0)

        for p in range(2):
            q = 2 * c + p
            qE = q * E
            w0 = wv[q, 0:16]
            deg_on = with_deg and p == 0

            def zrow(r, carry):
                sbuf[r, 0:16] = zero
                return carry
            jax.lax.fori_loop(0, KE, zrow, 0)

            def zacc(k, carry):
                pz = s + 16 * k

                @pl.when(pz < npz)
                def _():
                    pltpu.sync_copy(sbuf, acc.at[pl.ds(pz * KE, KE)])
                    if deg_on:
                        @pl.when(c == 0)
                        def _():
                            pltpu.sync_copy(zb1, dacc.at[pl.ds(pz * KE, KE)])
                return carry
            jax.lax.fori_loop(0, (npz + 15) // 16, zacc, 0)

            plsc.subcore_barrier()

            def issue_ab(ch, sl):
                base = s * EPT + ch * KE
                pltpu.async_copy(dstq_hbm.at[pl.ds(qE + base, KE)],
                                 iqa.at[sl], semi[sl])
                pltpu.async_copy(srcq_hbm.at[pl.ds(qE + base, KE)],
                                 iqb.at[sl], semi[sl])

            def wait_ab(sl):
                pltpu.make_async_copy(dstq_hbm.at[pl.ds(0, KE)], iqa.at[sl],
                                      semi[sl]).wait()
                pltpu.make_async_copy(srcq_hbm.at[pl.ds(0, KE)], iqb.at[sl],
                                      semi[sl]).wait()

            def issue_de(ch, sl):
                base = s * EPT + ch * KE
                pltpu.async_copy(dst_hbm.at[pl.ds(base, KE)], idv.at[sl],
                                 semE[sl])
                pltpu.async_copy(ea_hbm.at[pl.ds(base, KE)], iea.at[sl],
                                 semE[sl])

            def wait_de(sl):
                pltpu.make_async_copy(dst_hbm.at[pl.ds(0, KE)], idv.at[sl],
                                      semE[sl]).wait()
                pltpu.make_async_copy(ea_hbm.at[pl.ds(0, KE)], iea.at[sl],
                                      semE[sl]).wait()

            def issue_gather(sl):
                pltpu.async_copy(A_hbm.at[iqa.at[sl]], gA.at[sl], semA[sl])
                pltpu.async_copy(B_hbm.at[iqb.at[sl]], gB.at[sl], semB[sl])

            def wait_gather(sl):
                pltpu.make_async_copy(A_hbm.at[iqa.at[sl]], gA.at[sl],
                                      semA[sl]).wait()
                pltpu.make_async_copy(B_hbm.at[iqb.at[sl]], gB.at[sl],
                                      semB[sl]).wait()

            def half_step(ch, sl):
                wait_gather(sl)

                @pl.when(ch + 2 < NCH)
                def _():
                    issue_ab(ch + 2, sl)

                wait_de(sl)

                def edge16(g, cr):
                    e0 = g * 16
                    ev = iea[sl, pl.ds(e0, 16)]
                    for j in range(16):
                        e = e0 + j
                        t = gA[sl, e, 0:16] + gB[sl, e, 0:16] + ev[j] * w0
                        sbuf[e, 0:16] = t / (1.0 + jnp.exp(-t))
                    return cr
                jax.lax.fori_loop(0, KE // 16, edge16, 0)

                pltpu.sync_copy(sbuf, acc.at[idv.at[sl]], add=True)
                if deg_on:
                    @pl.when(c == 0)
                    def _():
                        pltpu.sync_copy(ones, dacc.at[idv.at[sl]], add=True)

                @pl.when(ch + 2 < NCH)
                def _():
                    issue_de(ch + 2, sl)
                    wait_ab(sl)
                    issue_gather(sl)

            issue_ab(0, 0)
            issue_de(0, 0)
            issue_ab(1, 1)
            issue_de(1, 1)
            wait_ab(0)
            issue_gather(0)
            wait_ab(1)
            issue_gather(1)

            def pair(i, carry):
                half_step(2 * i, 0)
                half_step(2 * i + 1, 1)
                return carry
            jax.lax.fori_loop(0, NCH // 2, pair, 0)
            if NCH % 2:
                half_step(NCH - 1, 0)

            plsc.subcore_barrier()

            def wout(k, carry):
                pz = s + 16 * k

                @pl.when(pz < npz)
                def _():
                    pltpu.sync_copy(acc.at[pl.ds(pz * KE, KE)],
                                    out_hbm.at[pl.ds(q * N + pz * KE, KE)])
                    if deg_on:
                        @pl.when(c == 0)
                        def _():
                            pltpu.sync_copy(dacc.at[pl.ds(pz * KE, KE)],
                                            dout_hbm.at[pl.ds(pz * KE, KE)])
                return carry
            jax.lax.fori_loop(0, (npz + 15) // 16, wout, 0)

            plsc.subcore_barrier()

    return pl.kernel(
        body,
        mesh=_SC_MESH,
        compiler_params=pltpu.CompilerParams(use_tc_tiling_on_sc=False),
        out_type=out_type if with_deg else out_type[0],
        scratch_types=scratch,
    )


_sc_edge_deg = _make_sc_edge(True)
_sc_edge = _make_sc_edge(False)


def kernel(x, edge_index, edge_attr, batch, enc_W, enc_b, mlp1_W, mlp1_b,
           mlp2_W, mlp2_b, ln_g, ln_b, head1_W, head1_b, head2_W, head2_b):
    src = edge_index[0]
    dst = edge_index[1]
    ea = edge_attr.reshape(E)
    L = mlp1_W.shape[0]

    qoff = (jnp.arange(4, dtype=jnp.int32) * N)[:, None]
    dstq = (dst[None, :] + qoff).reshape(4 * E)
    srcq = (src[None, :] + qoff).reshape(4 * E)

    def layer_weights(l):
        W1 = mlp1_W[l]
        W1ab = jnp.concatenate([W1[:H], W1[H:2 * H]], axis=1)
        b1ab = jnp.concatenate(
            [mlp1_b[l], jnp.zeros((H,), jnp.float32)]).reshape(1, 2 * H)
        w_st = W1[2 * H].reshape(4, HQ)
        return W1ab, b1ab, w_st

    W1ab0, b1ab0, w_st0 = layer_weights(0)
    h, Ast, Bst = _enc_pre(x, enc_W, enc_b, W1ab0, b1ab0)
    Sflat, degf = _sc_edge_deg(Ast.reshape(4 * N, HQ), Bst.reshape(4 * N, HQ),
                               dstq, srcq, dst, ea, w_st0)
    deg = degf.reshape(N, 1)

    for l in range(1, L):
        W1ab, b1ab, w_st = layer_weights(l)
        h, Ast, Bst = _post_pre(h, Sflat, deg, mlp2_W[l - 1], mlp2_b[l - 1],
                                ln_g[l - 1], ln_b[l - 1], W1ab, b1ab)
        Sflat = _sc_edge(Ast.reshape(4 * N, HQ), Bst.reshape(4 * N, HQ),
                         dstq, srcq, dst, ea, w_st)

    return _post_pool(h, Sflat, deg, mlp2_W[L - 1], mlp2_b[L - 1],
                      ln_g[L - 1], ln_b[L - 1], batch, head1_W, head1_b,
                      head2_W, head2_b)
